# baseline (device time: 390208 ns/iter reference)
import jax
import jax.numpy as jnp
from jax import lax
from jax.experimental import pallas as pl
from jax.experimental.pallas import tpu as pltpu

N_DEV = 16
HEADS_PER = 8
SQ = 2048
DH = 128
DM = 1024
QB = 512
WINDOW = 128
SCALE = 0.08838834764831843
CHUNK = SQ // N_DEV


def _attn_body(x_ref, wq_ref, k_ref, v_ref, wo_ref, out_ref):
    qb = pl.program_id(0)
    h = pl.program_id(1)

    q = jnp.dot(x_ref[...], wq_ref[0], preferred_element_type=jnp.float32)
    s = lax.dot_general(
        q, k_ref[0], (((1,), (1,)), ((), ())),
        preferred_element_type=jnp.float32,
    ) * SCALE
    qi = qb * QB + lax.broadcasted_iota(jnp.int32, (QB, SQ), 0)
    ki = lax.broadcasted_iota(jnp.int32, (QB, SQ), 1)
    s = jnp.where(jnp.abs(qi - ki) <= WINDOW, s, -1e9)
    m = jnp.max(s, axis=1, keepdims=True)
    e = jnp.exp(s - m)
    p = e / jnp.sum(e, axis=1, keepdims=True)
    ctx = jnp.dot(p, v_ref[0], preferred_element_type=jnp.float32)
    delta = jnp.dot(ctx, wo_ref[0], preferred_element_type=jnp.float32)

    @pl.when(h == 0)
    def _():
        out_ref[...] = delta

    @pl.when(h != 0)
    def _():
        out_ref[...] += delta


def _attn(x2, wq_h, k_h, v_h, wo_h):
    return pl.pallas_call(
        _attn_body,
        grid=(SQ // QB, HEADS_PER),
        in_specs=[
            pl.BlockSpec((QB, DM), lambda qb, h: (qb, 0)),
            pl.BlockSpec((1, DM, DH), lambda qb, h: (h, 0, 0)),
            pl.BlockSpec((1, SQ, DH), lambda qb, h: (h, 0, 0)),
            pl.BlockSpec((1, SQ, DH), lambda qb, h: (h, 0, 0)),
            pl.BlockSpec((1, DH, DM), lambda qb, h: (h, 0, 0)),
        ],
        out_specs=pl.BlockSpec((QB, DM), lambda qb, h: (qb, 0)),
        out_shape=jax.ShapeDtypeStruct((SQ, DM), jnp.float32),
        compiler_params=pltpu.CompilerParams(
            dimension_semantics=("arbitrary", "arbitrary"),
        ),
    )(x2, wq_h, k_h, v_h, wo_h)


def _allreduce_body(in_ref, out_ref, send_buf, recv_buf,
                    rs_send_sems, rs_recv_sems, ag_send_sems, ag_recv_sems):
    i = lax.axis_index("i")
    left = lax.rem(i + N_DEV - 1, N_DEV)
    right = lax.rem(i + 1, N_DEV)

    barrier_sem = pltpu.get_barrier_semaphore()
    for nbr in (left, right):
        pl.semaphore_signal(
            barrier_sem, inc=1,
            device_id=(nbr,), device_id_type=pl.DeviceIdType.MESH,
        )
    pl.semaphore_wait(barrier_sem, 2)

    for s in range(N_DEV - 1):
        c = lax.rem(i + N_DEV - s, N_DEV)
        if s == 0:
            send_buf[...] = in_ref[c]
        else:
            send_buf[...] = in_ref[c] + recv_buf[s - 1]
        rdma = pltpu.make_async_remote_copy(
            src_ref=send_buf,
            dst_ref=recv_buf.at[s],
            send_sem=rs_send_sems.at[s],
            recv_sem=rs_recv_sems.at[s],
            device_id=(right,),
            device_id_type=pl.DeviceIdType.MESH,
        )
        rdma.start()
        rdma.wait()

    c_mine = lax.rem(i + 1, N_DEV)
    out_ref[c_mine] = in_ref[c_mine] + recv_buf[N_DEV - 2]

    for s in range(N_DEV - 1):
        c = lax.rem(i + 1 + N_DEV - s, N_DEV)
        rdma = pltpu.make_async_remote_copy(
            src_ref=out_ref.at[c],
            dst_ref=out_ref.at[c],
            send_sem=ag_send_sems.at[s],
            recv_sem=ag_recv_sems.at[s],
            device_id=(right,),
            device_id_type=pl.DeviceIdType.MESH,
        )
        rdma.start()
        rdma.wait()


def _allreduce(partial):
    chunks = partial.reshape(N_DEV, CHUNK, DM)
    out = pl.pallas_call(
        _allreduce_body,
        in_specs=[pl.BlockSpec(memory_space=pltpu.VMEM)],
        out_specs=pl.BlockSpec(memory_space=pltpu.VMEM),
        out_shape=jax.ShapeDtypeStruct((N_DEV, CHUNK, DM), jnp.float32),
        scratch_shapes=[
            pltpu.VMEM((CHUNK, DM), jnp.float32),
            pltpu.VMEM((N_DEV - 1, CHUNK, DM), jnp.float32),
            pltpu.SemaphoreType.DMA((N_DEV - 1,)),
            pltpu.SemaphoreType.DMA((N_DEV - 1,)),
            pltpu.SemaphoreType.DMA((N_DEV - 1,)),
            pltpu.SemaphoreType.DMA((N_DEV - 1,)),
        ],
        compiler_params=pltpu.CompilerParams(collective_id=0),
    )(chunks)
    return out.reshape(SQ, DM)


def kernel(x, Wq, K_ext, V_ext, Wo):
    i = lax.axis_index("i")
    h0 = i * HEADS_PER

    x2 = x[0]
    k_h = lax.dynamic_slice_in_dim(K_ext[0], h0, HEADS_PER, axis=1)
    v_h = lax.dynamic_slice_in_dim(V_ext[0], h0, HEADS_PER, axis=1)
    k_h = jnp.transpose(k_h, (1, 0, 2))
    v_h = jnp.transpose(v_h, (1, 0, 2))
    wq_h = jnp.transpose(Wq.reshape(DM, HEADS_PER, DH), (1, 0, 2))
    wo_h = Wo.reshape(HEADS_PER, DH, DM)

    partial = _attn(x2, wq_h, k_h, v_h, wo_h)
    out = _allreduce(partial)
    return out[None]


# device time: 297485 ns/iter; 1.3117x vs baseline; 1.3117x over previous
import jax
import jax.numpy as jnp
from jax import lax
from jax.experimental import pallas as pl
from jax.experimental.pallas import tpu as pltpu

N_DEV = 16
HEADS_PER = 8
SQ = 2048
DH = 128
DM = 1024
QB = 512
KB = 768
WINDOW = 128
SCALE = 0.08838834764831843
CHUNK = SQ // N_DEV
HALF = DM // 2


def _attn_body(x_ref, wq_ref, k_ref, v_ref, wo_ref, out_ref):
    qb = pl.program_id(0)
    h = pl.program_id(1)

    q0 = qb * QB
    start = jnp.clip(q0 - WINDOW, 0, SQ - KB)
    k_win = k_ref[0, pl.ds(start, KB), :]
    v_win = v_ref[0, pl.ds(start, KB), :]

    q = jnp.dot(x_ref[...], wq_ref[0], preferred_element_type=jnp.float32)
    s = lax.dot_general(
        q, k_win, (((1,), (1,)), ((), ())),
        preferred_element_type=jnp.float32,
    ) * SCALE
    qi = q0 + lax.broadcasted_iota(jnp.int32, (QB, KB), 0)
    ki = start + lax.broadcasted_iota(jnp.int32, (QB, KB), 1)
    s = jnp.where(jnp.abs(qi - ki) <= WINDOW, s, -1e9)
    m = jnp.max(s, axis=1, keepdims=True)
    e = jnp.exp(s - m)
    p = e / jnp.sum(e, axis=1, keepdims=True)
    ctx = jnp.dot(p, v_win, preferred_element_type=jnp.float32)
    delta = jnp.dot(ctx, wo_ref[0], preferred_element_type=jnp.float32)

    @pl.when(h == 0)
    def _():
        out_ref[...] = delta

    @pl.when(h != 0)
    def _():
        out_ref[...] += delta


def _attn(x2, wq_h, k_h, v_h, wo_h):
    return pl.pallas_call(
        _attn_body,
        grid=(SQ // QB, HEADS_PER),
        in_specs=[
            pl.BlockSpec((QB, DM), lambda qb, h: (qb, 0)),
            pl.BlockSpec((1, DM, DH), lambda qb, h: (h, 0, 0)),
            pl.BlockSpec((1, SQ, DH), lambda qb, h: (h, 0, 0)),
            pl.BlockSpec((1, SQ, DH), lambda qb, h: (h, 0, 0)),
            pl.BlockSpec((1, DH, DM), lambda qb, h: (h, 0, 0)),
        ],
        out_specs=pl.BlockSpec((QB, DM), lambda qb, h: (qb, 0)),
        out_shape=jax.ShapeDtypeStruct((SQ, DM), jnp.float32),
        compiler_params=pltpu.CompilerParams(
            dimension_semantics=("arbitrary", "arbitrary"),
        ),
    )(x2, wq_h, k_h, v_h, wo_h)


def _allreduce_body(inr_ref, inl_ref, outr_ref, outl_ref,
                    sb_r, sb_l, rb_r, rb_l,
                    rs_ss_r, rs_rs_r, ag_ss_r, ag_rs_r,
                    rs_ss_l, rs_rs_l, ag_ss_l, ag_rs_l):
    i = lax.axis_index("i")
    left = lax.rem(i + N_DEV - 1, N_DEV)
    right = lax.rem(i + 1, N_DEV)

    barrier_sem = pltpu.get_barrier_semaphore()
    for nbr in (left, right):
        pl.semaphore_signal(
            barrier_sem, inc=1,
            device_id=(nbr,), device_id_type=pl.DeviceIdType.MESH,
        )
    pl.semaphore_wait(barrier_sem, 2)

    def _rdma(src, dst, ssem, rsem, dev):
        return pltpu.make_async_remote_copy(
            src_ref=src, dst_ref=dst, send_sem=ssem, recv_sem=rsem,
            device_id=(dev,), device_id_type=pl.DeviceIdType.MESH,
        )

    for s in range(N_DEV - 1):
        cr = lax.rem(i + N_DEV - s, N_DEV)
        cl = lax.rem(i + s, N_DEV)
        if s == 0:
            sb_r[...] = inr_ref[cr]
            sb_l[...] = inl_ref[cl]
        else:
            sb_r[...] = inr_ref[cr] + rb_r[s - 1]
            sb_l[...] = inl_ref[cl] + rb_l[s - 1]
        r = _rdma(sb_r, rb_r.at[s], rs_ss_r.at[s], rs_rs_r.at[s], right)
        l = _rdma(sb_l, rb_l.at[s], rs_ss_l.at[s], rs_rs_l.at[s], left)
        r.start()
        l.start()
        r.wait()
        l.wait()

    cr_mine = lax.rem(i + 1, N_DEV)
    cl_mine = lax.rem(i + N_DEV - 1, N_DEV)
    outr_ref[cr_mine] = inr_ref[cr_mine] + rb_r[N_DEV - 2]
    outl_ref[cl_mine] = inl_ref[cl_mine] + rb_l[N_DEV - 2]

    for s in range(N_DEV - 1):
        cr = lax.rem(i + 1 + N_DEV - s, N_DEV)
        cl = lax.rem(i + N_DEV - 1 + s, N_DEV)
        r = _rdma(outr_ref.at[cr], outr_ref.at[cr],
                  ag_ss_r.at[s], ag_rs_r.at[s], right)
        l = _rdma(outl_ref.at[cl], outl_ref.at[cl],
                  ag_ss_l.at[s], ag_rs_l.at[s], left)
        r.start()
        l.start()
        r.wait()
        l.wait()


def _allreduce(partial):
    in_r = partial[:, :HALF].reshape(N_DEV, CHUNK, HALF)
    in_l = partial[:, HALF:].reshape(N_DEV, CHUNK, HALF)
    out_r, out_l = pl.pallas_call(
        _allreduce_body,
        in_specs=[
            pl.BlockSpec(memory_space=pltpu.VMEM),
            pl.BlockSpec(memory_space=pltpu.VMEM),
        ],
        out_specs=[
            pl.BlockSpec(memory_space=pltpu.VMEM),
            pl.BlockSpec(memory_space=pltpu.VMEM),
        ],
        out_shape=[
            jax.ShapeDtypeStruct((N_DEV, CHUNK, HALF), jnp.float32),
            jax.ShapeDtypeStruct((N_DEV, CHUNK, HALF), jnp.float32),
        ],
        scratch_shapes=[
            pltpu.VMEM((CHUNK, HALF), jnp.float32),
            pltpu.VMEM((CHUNK, HALF), jnp.float32),
            pltpu.VMEM((N_DEV - 1, CHUNK, HALF), jnp.float32),
            pltpu.VMEM((N_DEV - 1, CHUNK, HALF), jnp.float32),
        ] + [pltpu.SemaphoreType.DMA((N_DEV - 1,)) for _ in range(8)],
        compiler_params=pltpu.CompilerParams(collective_id=0),
    )(in_r, in_l)
    out = jnp.concatenate(
        [out_r.reshape(SQ, HALF), out_l.reshape(SQ, HALF)], axis=1
    )
    return out


def kernel(x, Wq, K_ext, V_ext, Wo):
    i = lax.axis_index("i")
    h0 = i * HEADS_PER

    x2 = x[0]
    k_h = lax.dynamic_slice_in_dim(K_ext[0], h0, HEADS_PER, axis=1)
    v_h = lax.dynamic_slice_in_dim(V_ext[0], h0, HEADS_PER, axis=1)
    k_h = jnp.transpose(k_h, (1, 0, 2))
    v_h = jnp.transpose(v_h, (1, 0, 2))
    wq_h = jnp.transpose(Wq.reshape(DM, HEADS_PER, DH), (1, 0, 2))
    wo_h = Wo.reshape(HEADS_PER, DH, DM)

    partial = _attn(x2, wq_h, k_h, v_h, wo_h)
    out = _allreduce(partial)
    return out[None]


# device time: 191155 ns/iter; 2.0413x vs baseline; 1.5563x over previous
import jax
import jax.numpy as jnp
from jax import lax
from jax.experimental import pallas as pl
from jax.experimental.pallas import tpu as pltpu

N_DEV = 16
HEADS_PER = 8
SQ = 2048
DH = 128
DM = 1024
QB = 512
KB = 768
WINDOW = 128
SCALE = 0.08838834764831843
HALF = DM // 2
PC = SQ // 4
ZC = PC // 4

f32 = jnp.float32
bf16 = jnp.bfloat16


def _attn_body(x_ref, wq_ref, k_ref, v_ref, wo_ref, out_ref):
    qb = pl.program_id(0)
    h = pl.program_id(1)

    q0 = qb * QB
    start = pl.multiple_of(jnp.clip(q0 - WINDOW, 0, SQ - KB), WINDOW)
    k_win = k_ref[0, pl.ds(start, KB), :]
    v_win = v_ref[0, pl.ds(start, KB), :]

    q = jnp.dot(x_ref[...], wq_ref[0], preferred_element_type=f32)
    s = lax.dot_general(
        q.astype(bf16), k_win, (((1,), (1,)), ((), ())),
        preferred_element_type=f32,
    ) * SCALE
    qi = q0 + lax.broadcasted_iota(jnp.int32, (QB, KB), 0)
    ki = start + lax.broadcasted_iota(jnp.int32, (QB, KB), 1)
    s = jnp.where(jnp.abs(qi - ki) <= WINDOW, s, -1e9)
    m = jnp.max(s, axis=1, keepdims=True)
    e = jnp.exp(s - m)
    p = e / jnp.sum(e, axis=1, keepdims=True)
    ctx = jnp.dot(p.astype(bf16), v_win, preferred_element_type=f32)
    delta = jnp.dot(ctx.astype(bf16), wo_ref[0], preferred_element_type=f32)

    @pl.when(h == 0)
    def _():
        out_ref[...] = delta

    @pl.when(h != 0)
    def _():
        out_ref[...] += delta


def _attn(x2, wq_h, k_h, v_h, wo_h):
    return pl.pallas_call(
        _attn_body,
        grid=(SQ // QB, HEADS_PER),
        in_specs=[
            pl.BlockSpec((QB, DM), lambda qb, h: (qb, 0)),
            pl.BlockSpec((1, DM, DH), lambda qb, h: (h, 0, 0)),
            pl.BlockSpec((1, SQ, DH), lambda qb, h: (h, 0, 0)),
            pl.BlockSpec((1, SQ, DH), lambda qb, h: (h, 0, 0)),
            pl.BlockSpec((1, DH, DM), lambda qb, h: (h, 0, 0)),
        ],
        out_specs=pl.BlockSpec((QB, DM), lambda qb, h: (qb, 0)),
        out_shape=jax.ShapeDtypeStruct((SQ, DM), f32),
        compiler_params=pltpu.CompilerParams(
            dimension_semantics=("arbitrary", "arbitrary"),
        ),
    )(x2, wq_h, k_h, v_h, wo_h)


def _ar_body(inr_ref, inl_ref, outr_ref, outl_ref,
             sb_r, sb_l, rv1_r, rv1_l, mq_r, mq_l,
             zsb_r, zsb_l, zrv_r, zrv_l, zag_r, zag_l, pag_r, pag_l,
             p1s_r, p1r_r, p1s_l, p1r_l,
             zrs_s_r, zrs_r_r, zrs_s_l, zrs_r_l,
             zag_s_r, zag_r_r, zag_s_l, zag_r_l,
             p3s_r, p3r_r, p3s_l, p3r_l):
    i = lax.axis_index("i")
    p = lax.div(i, 4)
    j = lax.rem(i, 4)
    p4 = p * 4
    pright = p4 + lax.rem(j + 1, 4)
    pleft = p4 + lax.rem(j + 3, 4)
    zup = lax.rem(p + 1, 4) * 4 + j
    zdown = lax.rem(p + 3, 4) * 4 + j

    barrier_sem = pltpu.get_barrier_semaphore()
    for nbr in (pright, pleft, zup, zdown):
        pl.semaphore_signal(
            barrier_sem, inc=1,
            device_id=(nbr,), device_id_type=pl.DeviceIdType.MESH,
        )
    pl.semaphore_wait(barrier_sem, 4)

    def _rdma(src, dst, ssem, rsem, dev):
        return pltpu.make_async_remote_copy(
            src_ref=src, dst_ref=dst, send_sem=ssem, recv_sem=rsem,
            device_id=(dev,), device_id_type=pl.DeviceIdType.MESH,
        )

    for s in range(3):
        cr = lax.rem(j + 4 - s, 4)
        cl = lax.rem(j + s, 4)
        if s == 0:
            sb_r[...] = inr_ref[cr].astype(bf16)
            sb_l[...] = inl_ref[cl].astype(bf16)
        else:
            sb_r[...] = (inr_ref[cr] + rv1_r[s - 1].astype(f32)).astype(bf16)
            sb_l[...] = (inl_ref[cl] + rv1_l[s - 1].astype(f32)).astype(bf16)
        r = _rdma(sb_r, rv1_r.at[s], p1s_r.at[s], p1r_r.at[s], pright)
        l = _rdma(sb_l, rv1_l.at[s], p1s_l.at[s], p1r_l.at[s], pleft)
        r.start()
        l.start()
        r.wait()
        l.wait()

    qj_r = lax.rem(j + 1, 4)
    qj_l = lax.rem(j + 3, 4)
    mq_r[...] = inr_ref[qj_r] + rv1_r[2].astype(f32)
    mq_l[...] = inl_ref[qj_l] + rv1_l[2].astype(f32)

    for s in range(3):
        zr = lax.rem(p + 4 - s, 4)
        zl = lax.rem(p + s, 4)
        if s == 0:
            zsb_r[...] = mq_r[pl.ds(zr * ZC, ZC), :].astype(bf16)
            zsb_l[...] = mq_l[pl.ds(zl * ZC, ZC), :].astype(bf16)
        else:
            zsb_r[...] = (mq_r[pl.ds(zr * ZC, ZC), :]
                          + zrv_r[s - 1].astype(f32)).astype(bf16)
            zsb_l[...] = (mq_l[pl.ds(zl * ZC, ZC), :]
                          + zrv_l[s - 1].astype(f32)).astype(bf16)
        r = _rdma(zsb_r, zrv_r.at[s], zrs_s_r.at[s], zrs_r_r.at[s], zup)
        l = _rdma(zsb_l, zrv_l.at[s], zrs_s_l.at[s], zrs_r_l.at[s], zdown)
        r.start()
        l.start()
        r.wait()
        l.wait()

    zp_r = lax.rem(p + 1, 4)
    zp_l = lax.rem(p + 3, 4)
    zag_r[zp_r] = (mq_r[pl.ds(zp_r * ZC, ZC), :]
                   + zrv_r[2].astype(f32)).astype(bf16)
    zag_l[zp_l] = (mq_l[pl.ds(zp_l * ZC, ZC), :]
                   + zrv_l[2].astype(f32)).astype(bf16)

    for s in range(3):
        slot_r = lax.rem(p + 1 + 4 - s, 4)
        slot_l = lax.rem(p + 3 + s, 4)
        r = _rdma(zag_r.at[slot_r], zag_r.at[slot_r],
                  zag_s_r.at[s], zag_r_r.at[s], zup)
        l = _rdma(zag_l.at[slot_l], zag_l.at[slot_l],
                  zag_s_l.at[s], zag_r_l.at[s], zdown)
        r.start()
        l.start()
        r.wait()
        l.wait()

    pag_r[qj_r] = zag_r[...].reshape(PC, HALF)
    pag_l[qj_l] = zag_l[...].reshape(PC, HALF)

    for s in range(3):
        slot_r = lax.rem(j + 1 + 4 - s, 4)
        slot_l = lax.rem(j + 3 + s, 4)
        r = _rdma(pag_r.at[slot_r], pag_r.at[slot_r],
                  p3s_r.at[s], p3r_r.at[s], pright)
        l = _rdma(pag_l.at[slot_l], pag_l.at[slot_l],
                  p3s_l.at[s], p3r_l.at[s], pleft)
        r.start()
        l.start()
        r.wait()
        l.wait()

    outr_ref[...] = pag_r[...].astype(f32)
    outl_ref[...] = pag_l[...].astype(f32)


def _allreduce(partial):
    in_r = partial[:, :HALF].reshape(4, PC, HALF)
    in_l = partial[:, HALF:].reshape(4, PC, HALF)
    sems = [pltpu.SemaphoreType.DMA((3,)) for _ in range(16)]
    out_r, out_l = pl.pallas_call(
        _ar_body,
        in_specs=[
            pl.BlockSpec(memory_space=pltpu.VMEM),
            pl.BlockSpec(memory_space=pltpu.VMEM),
        ],
        out_specs=[
            pl.BlockSpec(memory_space=pltpu.VMEM),
            pl.BlockSpec(memory_space=pltpu.VMEM),
        ],
        out_shape=[
            jax.ShapeDtypeStruct((4, PC, HALF), f32),
            jax.ShapeDtypeStruct((4, PC, HALF), f32),
        ],
        scratch_shapes=[
            pltpu.VMEM((PC, HALF), bf16),
            pltpu.VMEM((PC, HALF), bf16),
            pltpu.VMEM((3, PC, HALF), bf16),
            pltpu.VMEM((3, PC, HALF), bf16),
            pltpu.VMEM((PC, HALF), f32),
            pltpu.VMEM((PC, HALF), f32),
            pltpu.VMEM((ZC, HALF), bf16),
            pltpu.VMEM((ZC, HALF), bf16),
            pltpu.VMEM((3, ZC, HALF), bf16),
            pltpu.VMEM((3, ZC, HALF), bf16),
            pltpu.VMEM((4, ZC, HALF), bf16),
            pltpu.VMEM((4, ZC, HALF), bf16),
            pltpu.VMEM((4, PC, HALF), bf16),
            pltpu.VMEM((4, PC, HALF), bf16),
        ] + sems,
        compiler_params=pltpu.CompilerParams(collective_id=0),
    )(in_r, in_l)
    out = jnp.concatenate(
        [out_r.reshape(SQ, HALF), out_l.reshape(SQ, HALF)], axis=1
    )
    return out


def kernel(x, Wq, K_ext, V_ext, Wo):
    i = lax.axis_index("i")
    h0 = i * HEADS_PER

    x2 = x[0].astype(bf16)
    k_h = lax.dynamic_slice_in_dim(K_ext[0], h0, HEADS_PER, axis=1)
    v_h = lax.dynamic_slice_in_dim(V_ext[0], h0, HEADS_PER, axis=1)
    k_h = jnp.transpose(k_h, (1, 0, 2)).astype(bf16)
    v_h = jnp.transpose(v_h, (1, 0, 2)).astype(bf16)
    wq_h = jnp.transpose(Wq.reshape(DM, HEADS_PER, DH), (1, 0, 2)).astype(bf16)
    wo_h = Wo.reshape(HEADS_PER, DH, DM).astype(bf16)

    partial = _attn(x2, wq_h, k_h, v_h, wo_h)
    out = _allreduce(partial)
    return out[None]


# device time: 177644 ns/iter; 2.1966x vs baseline; 1.0761x over previous
import jax
import jax.numpy as jnp
from jax import lax
from jax.experimental import pallas as pl
from jax.experimental.pallas import tpu as pltpu

N_DEV = 16
HEADS_PER = 8
SQ = 2048
DH = 128
DM = 1024
QB = 256
KB = 512
WINDOW = 128
SCALE = 0.08838834764831843
HALF = DM // 2
PC = SQ // 4
ZC = PC // 4

f32 = jnp.float32
bf16 = jnp.bfloat16


def _attn_body(x_ref, wq_ref, k_ref, v_ref, wo_ref, out_ref):
    qb = pl.program_id(0)
    h = pl.program_id(1)

    q0 = qb * QB
    start = pl.multiple_of(jnp.clip(q0 - WINDOW, 0, SQ - KB), WINDOW)
    k_win = k_ref[0, pl.ds(start, KB), :]
    v_win = v_ref[0, pl.ds(start, KB), :]

    q = jnp.dot(x_ref[...], wq_ref[0], preferred_element_type=f32)
    s = lax.dot_general(
        q.astype(bf16), k_win, (((1,), (1,)), ((), ())),
        preferred_element_type=f32,
    ) * SCALE
    qi = q0 + lax.broadcasted_iota(jnp.int32, (QB, KB), 0)
    ki = start + lax.broadcasted_iota(jnp.int32, (QB, KB), 1)
    e = jnp.exp(jnp.where(jnp.abs(qi - ki) <= WINDOW, s, -1e9))
    p = e / jnp.sum(e, axis=1, keepdims=True)
    ctx = jnp.dot(p.astype(bf16), v_win, preferred_element_type=f32)
    delta = jnp.dot(ctx.astype(bf16), wo_ref[0], preferred_element_type=f32)

    @pl.when(h == 0)
    def _():
        out_ref[...] = delta

    @pl.when(h != 0)
    def _():
        out_ref[...] += delta


def _attn(x2, wq_h, k_h, v_h, wo_h):
    return pl.pallas_call(
        _attn_body,
        grid=(SQ // QB, HEADS_PER),
        in_specs=[
            pl.BlockSpec((QB, DM), lambda qb, h: (qb, 0)),
            pl.BlockSpec((1, DM, DH), lambda qb, h: (h, 0, 0)),
            pl.BlockSpec((1, SQ, DH), lambda qb, h: (h, 0, 0)),
            pl.BlockSpec((1, SQ, DH), lambda qb, h: (h, 0, 0)),
            pl.BlockSpec((1, DH, DM), lambda qb, h: (h, 0, 0)),
        ],
        out_specs=pl.BlockSpec((QB, DM), lambda qb, h: (qb, 0)),
        out_shape=jax.ShapeDtypeStruct((SQ, DM), f32),
        compiler_params=pltpu.CompilerParams(
            dimension_semantics=("arbitrary", "arbitrary"),
        ),
    )(x2, wq_h, k_h, v_h, wo_h)


def _ar_body(inr_ref, inl_ref, outr_ref, outl_ref,
             sb_r, sb_l, rv1_r, rv1_l, mq_r, mq_l,
             zsb_r, zsb_l, zrv_r, zrv_l, zag_r, zag_l, pag_r, pag_l,
             p1s_r, p1r_r, p1s_l, p1r_l,
             zrs_s_r, zrs_r_r, zrs_s_l, zrs_r_l,
             zag_s_r, zag_r_r, zag_s_l, zag_r_l,
             p3s_r, p3r_r, p3s_l, p3r_l):
    i = lax.axis_index("i")
    p = lax.div(i, 4)
    j = lax.rem(i, 4)
    p4 = p * 4
    pright = p4 + lax.rem(j + 1, 4)
    pleft = p4 + lax.rem(j + 3, 4)
    zup = lax.rem(p + 1, 4) * 4 + j
    zdown = lax.rem(p + 3, 4) * 4 + j

    barrier_sem = pltpu.get_barrier_semaphore()
    for nbr in (pright, pleft, zup, zdown):
        pl.semaphore_signal(
            barrier_sem, inc=1,
            device_id=(nbr,), device_id_type=pl.DeviceIdType.MESH,
        )
    pl.semaphore_wait(barrier_sem, 4)

    def _rdma(src, dst, ssem, rsem, dev):
        return pltpu.make_async_remote_copy(
            src_ref=src, dst_ref=dst, send_sem=ssem, recv_sem=rsem,
            device_id=(dev,), device_id_type=pl.DeviceIdType.MESH,
        )

    for s in range(3):
        cr = lax.rem(j + 4 - s, 4)
        cl = lax.rem(j + s, 4)
        if s == 0:
            sb_r[...] = inr_ref[cr].astype(bf16)
            sb_l[...] = inl_ref[cl].astype(bf16)
        else:
            sb_r[...] = (inr_ref[cr] + rv1_r[s - 1].astype(f32)).astype(bf16)
            sb_l[...] = (inl_ref[cl] + rv1_l[s - 1].astype(f32)).astype(bf16)
        r = _rdma(sb_r, rv1_r.at[s], p1s_r.at[s], p1r_r.at[s], pright)
        l = _rdma(sb_l, rv1_l.at[s], p1s_l.at[s], p1r_l.at[s], pleft)
        r.start()
        l.start()
        r.wait()
        l.wait()

    qj_r = lax.rem(j + 1, 4)
    qj_l = lax.rem(j + 3, 4)
    mq_r[...] = inr_ref[qj_r] + rv1_r[2].astype(f32)
    mq_l[...] = inl_ref[qj_l] + rv1_l[2].astype(f32)

    for s in range(3):
        zr = lax.rem(p + 4 - s, 4)
        zl = lax.rem(p + s, 4)
        if s == 0:
            zsb_r[...] = mq_r[pl.ds(zr * ZC, ZC), :].astype(bf16)
            zsb_l[...] = mq_l[pl.ds(zl * ZC, ZC), :].astype(bf16)
        else:
            zsb_r[...] = (mq_r[pl.ds(zr * ZC, ZC), :]
                          + zrv_r[s - 1].astype(f32)).astype(bf16)
            zsb_l[...] = (mq_l[pl.ds(zl * ZC, ZC), :]
                          + zrv_l[s - 1].astype(f32)).astype(bf16)
        r = _rdma(zsb_r, zrv_r.at[s], zrs_s_r.at[s], zrs_r_r.at[s], zup)
        l = _rdma(zsb_l, zrv_l.at[s], zrs_s_l.at[s], zrs_r_l.at[s], zdown)
        r.start()
        l.start()
        r.wait()
        l.wait()

    zp_r = lax.rem(p + 1, 4)
    zp_l = lax.rem(p + 3, 4)
    zag_r[zp_r] = (mq_r[pl.ds(zp_r * ZC, ZC), :]
                   + zrv_r[2].astype(f32)).astype(bf16)
    zag_l[zp_l] = (mq_l[pl.ds(zp_l * ZC, ZC), :]
                   + zrv_l[2].astype(f32)).astype(bf16)

    for s in range(3):
        slot_r = lax.rem(p + 1 + 4 - s, 4)
        slot_l = lax.rem(p + 3 + s, 4)
        r = _rdma(zag_r.at[slot_r], zag_r.at[slot_r],
                  zag_s_r.at[s], zag_r_r.at[s], zup)
        l = _rdma(zag_l.at[slot_l], zag_l.at[slot_l],
                  zag_s_l.at[s], zag_r_l.at[s], zdown)
        r.start()
        l.start()
        r.wait()
        l.wait()

    pag_r[qj_r] = zag_r[...].reshape(PC, HALF)
    pag_l[qj_l] = zag_l[...].reshape(PC, HALF)

    for s in range(3):
        slot_r = lax.rem(j + 1 + 4 - s, 4)
        slot_l = lax.rem(j + 3 + s, 4)
        r = _rdma(pag_r.at[slot_r], pag_r.at[slot_r],
                  p3s_r.at[s], p3r_r.at[s], pright)
        l = _rdma(pag_l.at[slot_l], pag_l.at[slot_l],
                  p3s_l.at[s], p3r_l.at[s], pleft)
        r.start()
        l.start()
        r.wait()
        l.wait()

    outr_ref[...] = pag_r[...].astype(f32)
    outl_ref[...] = pag_l[...].astype(f32)


def _allreduce(partial):
    in_r = partial[:, :HALF].reshape(4, PC, HALF)
    in_l = partial[:, HALF:].reshape(4, PC, HALF)
    sems = [pltpu.SemaphoreType.DMA((3,)) for _ in range(16)]
    out_r, out_l = pl.pallas_call(
        _ar_body,
        in_specs=[
            pl.BlockSpec(memory_space=pltpu.VMEM),
            pl.BlockSpec(memory_space=pltpu.VMEM),
        ],
        out_specs=[
            pl.BlockSpec(memory_space=pltpu.VMEM),
            pl.BlockSpec(memory_space=pltpu.VMEM),
        ],
        out_shape=[
            jax.ShapeDtypeStruct((4, PC, HALF), f32),
            jax.ShapeDtypeStruct((4, PC, HALF), f32),
        ],
        scratch_shapes=[
            pltpu.VMEM((PC, HALF), bf16),
            pltpu.VMEM((PC, HALF), bf16),
            pltpu.VMEM((3, PC, HALF), bf16),
            pltpu.VMEM((3, PC, HALF), bf16),
            pltpu.VMEM((PC, HALF), f32),
            pltpu.VMEM((PC, HALF), f32),
            pltpu.VMEM((ZC, HALF), bf16),
            pltpu.VMEM((ZC, HALF), bf16),
            pltpu.VMEM((3, ZC, HALF), bf16),
            pltpu.VMEM((3, ZC, HALF), bf16),
            pltpu.VMEM((4, ZC, HALF), bf16),
            pltpu.VMEM((4, ZC, HALF), bf16),
            pltpu.VMEM((4, PC, HALF), bf16),
            pltpu.VMEM((4, PC, HALF), bf16),
        ] + sems,
        compiler_params=pltpu.CompilerParams(collective_id=0),
    )(in_r, in_l)
    out = jnp.concatenate(
        [out_r.reshape(SQ, HALF), out_l.reshape(SQ, HALF)], axis=1
    )
    return out


def kernel(x, Wq, K_ext, V_ext, Wo):
    i = lax.axis_index("i")
    h0 = i * HEADS_PER

    x2 = x[0].astype(bf16)
    k_h = lax.dynamic_slice_in_dim(K_ext[0], h0, HEADS_PER, axis=1)
    v_h = lax.dynamic_slice_in_dim(V_ext[0], h0, HEADS_PER, axis=1)
    k_h = jnp.transpose(k_h, (1, 0, 2)).astype(bf16)
    v_h = jnp.transpose(v_h, (1, 0, 2)).astype(bf16)
    wq_h = jnp.transpose(Wq.reshape(DM, HEADS_PER, DH), (1, 0, 2)).astype(bf16)
    wo_h = Wo.reshape(HEADS_PER, DH, DM).astype(bf16)

    partial = _attn(x2, wq_h, k_h, v_h, wo_h)
    out = _allreduce(partial)
    return out[None]
